# baseline (speedup 1.0000x reference)
import functools

import jax
import jax.numpy as jnp
from jax import lax
from jax.experimental import pallas as pl
from jax.experimental.pallas import tpu as pltpu
from jax.experimental.pallas import tpu_sc as plsc

B = 1024
N = 200
K = 50
D = 128

NC = 2
NS = 16
NW = NC * NS
BPW = B // NW
NBUF = 8
KP = 56
IPW = BPW * KP


def _build_sc_gather():
    mesh = plsc.VectorSubcoreMesh(core_axis_name="c", subcore_axis_name="s")

    @functools.partial(
        pl.kernel,
        mesh=mesh,
        out_type=jax.ShapeDtypeStruct((B, K, D), jnp.float32),
        scratch_types=[
            pltpu.VMEM((IPW,), jnp.int32),
        ] + [pltpu.VMEM((KP, D), jnp.float32) for _ in range(NBUF)]
          + [pltpu.SemaphoreType.DMA for _ in range(2 * NBUF)],
    )
    def sc_gather(table_hbm, idxp_hbm, out_hbm, flat_v, *bufs_and_sems):
        bufs = bufs_and_sems[:NBUF]
        gsems = bufs_and_sems[NBUF:2 * NBUF]
        wsems = bufs_and_sems[2 * NBUF:]
        wid = lax.axis_index("s") * NC + lax.axis_index("c")

        pltpu.sync_copy(idxp_hbm.at[pl.ds(wid * IPW, IPW)], flat_v)

        gh = [None] * NBUF
        wh = [None] * NBUF
        for j in range(min(NBUF, BPW)):
            gh[j] = pltpu.async_copy(
                table_hbm.at[flat_v.at[pl.ds(j * KP, KP)]], bufs[j], gsems[j])
        for j in range(BPW):
            b = j % NBUF
            gh[b].wait()
            wh[b] = pltpu.async_copy(
                bufs[b].at[pl.ds(0, K)], out_hbm.at[wid * BPW + j], wsems[b])
            nj = j + NBUF
            if nj < BPW:
                wh[b].wait()
                gh[b] = pltpu.async_copy(
                    table_hbm.at[flat_v.at[pl.ds(nj * KP, KP)]], bufs[b],
                    gsems[b])
        for j in range(max(0, BPW - NBUF), BPW):
            wh[j % NBUF].wait()

    return sc_gather


_sc_gather = _build_sc_gather()


def kernel(input_tensor, indices):
    table = input_tensor.reshape(B * N, D)
    idx = indices.astype(jnp.int32)
    off = (jnp.arange(B, dtype=jnp.int32) * N)[:, None]
    idxp = jnp.pad(idx + off, ((0, 0), (0, KP - K))).reshape(B * KP)
    return _sc_gather(table, idxp)


# speedup vs baseline: 1.0002x; 1.0002x over previous
import functools

import jax
import jax.numpy as jnp
from jax import lax
from jax.experimental import pallas as pl
from jax.experimental.pallas import tpu as pltpu
from jax.experimental.pallas import tpu_sc as plsc

B = 1024
N = 200
K = 50
D = 128

NC = 2
NS = 16
NW = NC * NS
BPW = B // NW
NBUF = 8
KP = 56
IPW = BPW * KP


def _build_sc_gather():
    mesh = plsc.VectorSubcoreMesh(core_axis_name="c", subcore_axis_name="s")

    @functools.partial(
        pl.kernel,
        mesh=mesh,
        out_type=jax.ShapeDtypeStruct((B, K, D), jnp.float32),
        scratch_types=[
            pltpu.VMEM((BPW, KP), jnp.int32),
        ] + [pltpu.VMEM((KP, D), jnp.float32) for _ in range(NBUF)]
          + [pltpu.SemaphoreType.DMA for _ in range(2 * NBUF)],
    )
    def sc_gather(table_hbm, idxp_hbm, out_hbm, flat_v, *bufs_and_sems):
        bufs = bufs_and_sems[:NBUF]
        gsems = bufs_and_sems[NBUF:2 * NBUF]
        wsems = bufs_and_sems[2 * NBUF:]
        wid = lax.axis_index("s") * NC + lax.axis_index("c")

        pltpu.sync_copy(idxp_hbm.at[wid], flat_v)

        gh = [None] * NBUF
        wh = [None] * NBUF
        for j in range(min(NBUF, BPW)):
            gh[j] = pltpu.async_copy(
                table_hbm.at[flat_v.at[j]], bufs[j], gsems[j])
        for j in range(BPW):
            b = j % NBUF
            gh[b].wait()
            wh[b] = pltpu.async_copy(
                bufs[b].at[pl.ds(0, K)], out_hbm.at[wid * BPW + j], wsems[b])
            nj = j + NBUF
            if nj < BPW:
                wh[b].wait()
                gh[b] = pltpu.async_copy(
                    table_hbm.at[flat_v.at[nj]], bufs[b], gsems[b])
        for j in range(max(0, BPW - NBUF), BPW):
            wh[j % NBUF].wait()

    return sc_gather


_sc_gather = _build_sc_gather()


def kernel(input_tensor, indices):
    table = input_tensor.reshape(B * N, D)
    idx = indices.astype(jnp.int32)
    off = (jnp.arange(B, dtype=jnp.int32) * N)[:, None]
    idxp = jnp.pad(idx + off, ((0, 0), (0, KP - K))).reshape(NW, BPW, KP)
    return _sc_gather(table, idxp)


# speedup vs baseline: 5.2101x; 5.2089x over previous
import functools

import jax
import jax.numpy as jnp
from jax import lax
from jax.experimental import pallas as pl
from jax.experimental.pallas import tpu as pltpu
from jax.experimental.pallas import tpu_sc as plsc

B = 1024
N = 200
K = 50
D = 128

NC = 2
NS = 16
NW = NC * NS
BPW = B // NW
NBUF = 8
KP = 56
IPW = BPW * KP


def _build_sc_gather():
    mesh = plsc.VectorSubcoreMesh(core_axis_name="c", subcore_axis_name="s")

    @functools.partial(
        pl.kernel,
        mesh=mesh,
        out_type=jax.ShapeDtypeStruct((B, K, D), jnp.float32),
        scratch_types=[
            pltpu.VMEM((BPW, KP), jnp.int32),
        ] + [pltpu.VMEM((KP, D), jnp.float32) for _ in range(NBUF)]
          + [pltpu.SemaphoreType.DMA for _ in range(2 * NBUF)],
    )
    def sc_gather(table_hbm, idxp_hbm, out_hbm, flat_v, *bufs_and_sems):
        bufs = bufs_and_sems[:NBUF]
        gsems = bufs_and_sems[NBUF:2 * NBUF]
        wsems = bufs_and_sems[2 * NBUF:]
        wid = lax.axis_index("s") * NC + lax.axis_index("c")

        pltpu.sync_copy(idxp_hbm.at[wid], flat_v)

        gh = [None] * NBUF
        wh = [None] * NBUF
        for j in range(min(NBUF, BPW)):
            gh[j] = pltpu.async_copy(
                table_hbm.at[flat_v.at[j]], bufs[j], gsems[j])
        for j in range(BPW):
            b = j % NBUF
            gh[b].wait()
            wh[b] = pltpu.async_copy(
                bufs[b].at[pl.ds(0, K)], out_hbm.at[wid * BPW + j], wsems[b])
            nj = j + NBUF
            if nj < BPW:
                wh[b].wait()
                gh[b] = pltpu.async_copy(
                    table_hbm.at[flat_v.at[nj]], bufs[b], gsems[b])
        for j in range(max(0, BPW - NBUF), BPW):
            wh[j % NBUF].wait()

    return sc_gather


_sc_gather = _build_sc_gather()


def kernel(input_tensor, indices):
    table = input_tensor.reshape(B * N, D)
    idx = indices.astype(jnp.int32)
    off = (jnp.arange(B, dtype=jnp.int32) * N)[:, None]
    flat = idx + off
    idxp = jnp.concatenate([flat, flat[:, :KP - K]], axis=1)
    return _sc_gather(table, idxp.reshape(NW, BPW, KP))


# speedup vs baseline: 8.2013x; 1.5741x over previous
import functools

import jax
import jax.numpy as jnp
from jax import lax
from jax.experimental import pallas as pl
from jax.experimental.pallas import tpu as pltpu
from jax.experimental.pallas import tpu_sc as plsc

B = 1024
N = 200
K = 50
D = 128

NC = 2
NS = 16
NW = NC * NS
ROWS = B * K
RPW = ROWS // NW
CH = 80
NCH = RPW // CH
NBUF = 8


def _build_sc_gather():
    mesh = plsc.VectorSubcoreMesh(core_axis_name="c", subcore_axis_name="s")

    @functools.partial(
        pl.kernel,
        mesh=mesh,
        out_type=jax.ShapeDtypeStruct((K * B, D), jnp.float32),
        scratch_types=[
            pltpu.VMEM((NCH, CH), jnp.int32),
            pltpu.VMEM((NCH, CH), jnp.int32),
        ] + [pltpu.VMEM((CH, D), jnp.float32) for _ in range(NBUF)]
          + [pltpu.SemaphoreType.DMA for _ in range(2 * NBUF)],
    )
    def sc_gather(table_hbm, gid_hbm, sid_hbm, out_hbm,
                  gid_v, sid_v, *bufs_and_sems):
        bufs = bufs_and_sems[:NBUF]
        gsems = bufs_and_sems[NBUF:2 * NBUF]
        wsems = bufs_and_sems[2 * NBUF:]
        wid = lax.axis_index("s") * NC + lax.axis_index("c")

        pltpu.sync_copy(gid_hbm.at[wid], gid_v)
        pltpu.sync_copy(sid_hbm.at[wid], sid_v)

        gh = [None] * NBUF
        wh = [None] * NBUF
        for j in range(min(NBUF, NCH)):
            gh[j] = pltpu.async_copy(
                table_hbm.at[gid_v.at[j]], bufs[j], gsems[j])
        for j in range(NCH):
            b = j % NBUF
            gh[b].wait()
            wh[b] = pltpu.async_copy(
                bufs[b], out_hbm.at[sid_v.at[j]], wsems[b])
            nj = j + NBUF
            if nj < NCH:
                wh[b].wait()
                gh[b] = pltpu.async_copy(
                    table_hbm.at[gid_v.at[nj]], bufs[b], gsems[b])
        for j in range(max(0, NCH - NBUF), NCH):
            wh[j % NBUF].wait()

    return sc_gather


_sc_gather = _build_sc_gather()


def kernel(input_tensor, indices):
    table = input_tensor.reshape(B * N, D)
    idx = indices.astype(jnp.int32)
    off = (jnp.arange(B, dtype=jnp.int32) * N)[:, None]
    gid = (idx + off).reshape(NW, NCH, CH)
    sid = (
        jnp.arange(K, dtype=jnp.int32)[None, :] * B
        + jnp.arange(B, dtype=jnp.int32)[:, None]
    ).reshape(NW, NCH, CH)
    out = _sc_gather(table, gid, sid)
    return out.reshape(K, B, D).transpose(1, 0, 2)


# speedup vs baseline: 8.4529x; 1.0307x over previous
import functools

import jax
import jax.numpy as jnp
from jax import lax
from jax.experimental import pallas as pl
from jax.experimental.pallas import tpu as pltpu
from jax.experimental.pallas import tpu_sc as plsc

B = 1024
N = 200
K = 50
D = 128

NC = 2
NS = 16
NW = NC * NS
ROWS = B * K
RPW = ROWS // NW
CH = 80
NCH = RPW // CH
NBUF = 8


def _build_sc_gather():
    mesh = plsc.VectorSubcoreMesh(core_axis_name="c", subcore_axis_name="s")

    @functools.partial(
        pl.kernel,
        mesh=mesh,
        out_type=jax.ShapeDtypeStruct((K * B, D), jnp.float32),
        scratch_types=[
            pltpu.VMEM((NCH, CH), jnp.int32),
        ] + [pltpu.VMEM((CH, D), jnp.float32) for _ in range(NBUF)]
          + [pltpu.SemaphoreType.DMA for _ in range(2 * NBUF)],
    )
    def sc_gather(table_hbm, gid_hbm, out_hbm, gid_v, *bufs_and_sems):
        bufs = bufs_and_sems[:NBUF]
        gsems = bufs_and_sems[NBUF:2 * NBUF]
        wsems = bufs_and_sems[2 * NBUF:]
        wid = lax.axis_index("s") * NC + lax.axis_index("c")
        base = wid * RPW

        pltpu.sync_copy(gid_hbm.at[wid], gid_v)

        gh = [None] * NBUF
        wh = [None] * NBUF
        for j in range(min(NBUF, NCH)):
            gh[j] = pltpu.async_copy(
                table_hbm.at[gid_v.at[j]], bufs[j], gsems[j])
        for j in range(NCH):
            b = j % NBUF
            gh[b].wait()
            wh[b] = pltpu.async_copy(
                bufs[b], out_hbm.at[pl.ds(base + j * CH, CH)], wsems[b])
            nj = j + NBUF
            if nj < NCH:
                wh[b].wait()
                gh[b] = pltpu.async_copy(
                    table_hbm.at[gid_v.at[nj]], bufs[b], gsems[b])
        for j in range(max(0, NCH - NBUF), NCH):
            wh[j % NBUF].wait()

    return sc_gather


_sc_gather = _build_sc_gather()


def kernel(input_tensor, indices):
    table = input_tensor.reshape(B * N, D)
    idx = indices.astype(jnp.int32)
    off = (jnp.arange(B, dtype=jnp.int32) * N)[:, None]
    gid = (idx + off).T.reshape(NW, NCH, CH)
    out = _sc_gather(table, gid)
    return out.reshape(K, B, D).transpose(1, 0, 2)


# speedup vs baseline: 8.6152x; 1.0192x over previous
import functools

import jax
import jax.numpy as jnp
from jax import lax
from jax.experimental import pallas as pl
from jax.experimental.pallas import tpu as pltpu
from jax.experimental.pallas import tpu_sc as plsc

B = 1024
N = 200
K = 50
D = 128

NC = 2
NS = 16
NW = NC * NS
ROWS = B * K
RPW = ROWS // NW
CH = 80
NCH = RPW // CH
NBUF = 12


def _build_sc_gather():
    mesh = plsc.VectorSubcoreMesh(core_axis_name="c", subcore_axis_name="s")

    @functools.partial(
        pl.kernel,
        mesh=mesh,
        out_type=jax.ShapeDtypeStruct((K * B, D), jnp.float32),
        scratch_types=[
            pltpu.VMEM((NCH, CH), jnp.int32),
        ] + [pltpu.VMEM((CH, D), jnp.float32) for _ in range(NBUF)]
          + [pltpu.SemaphoreType.DMA for _ in range(2 * NBUF)],
    )
    def sc_gather(table_hbm, gid_hbm, out_hbm, gid_v, *bufs_and_sems):
        bufs = bufs_and_sems[:NBUF]
        gsems = bufs_and_sems[NBUF:2 * NBUF]
        wsems = bufs_and_sems[2 * NBUF:]
        wid = lax.axis_index("s") * NC + lax.axis_index("c")
        base = wid * RPW

        pltpu.sync_copy(gid_hbm.at[wid], gid_v)

        gh = [None] * NBUF
        wh = [None] * NBUF
        for j in range(min(NBUF, NCH)):
            gh[j] = pltpu.async_copy(
                table_hbm.at[gid_v.at[j]], bufs[j], gsems[j])
        for j in range(NCH):
            b = j % NBUF
            gh[b].wait()
            wh[b] = pltpu.async_copy(
                bufs[b], out_hbm.at[pl.ds(base + j * CH, CH)], wsems[b])
            nj = j + NBUF
            if nj < NCH:
                wh[b].wait()
                gh[b] = pltpu.async_copy(
                    table_hbm.at[gid_v.at[nj]], bufs[b], gsems[b])
        for j in range(max(0, NCH - NBUF), NCH):
            wh[j % NBUF].wait()

    return sc_gather


_sc_gather = _build_sc_gather()


def kernel(input_tensor, indices):
    table = input_tensor.reshape(B * N, D)
    idx = indices.astype(jnp.int32)
    off = (jnp.arange(B, dtype=jnp.int32) * N)[:, None]
    gid = (idx + off).T.reshape(NW, NCH, CH)
    out = _sc_gather(table, gid)
    return out.reshape(K, B, D).transpose(1, 0, 2)
